# baseline (speedup 1.0000x reference)
import functools

import jax
import jax.numpy as jnp
from jax import lax
from jax.experimental import pallas as pl
from jax.experimental.pallas import tpu as pltpu
from jax.experimental.pallas import tpu_sc as plsc

B = 2
S = 2048
D = 1024
H = 4
HD = 64
IDX_D = H * HD
K_TOP = 256
HALF = D // 2
BT = 512
_LN1E4 = 9.210340371976184

_NC = 2
_NS = 16
_NW = _NC * _NS
_ROWS = B * K_TOP
_RPW = _ROWS // _NW


def _proj_body(x16_ref, x_ref, wqkv_ref, bqkv_ref, wiq_ref, wik_ref,
               q_ref, k_ref, v_ref, qi_ref, ki_ref):
    t = pl.program_id(1)
    xb = x_ref[0]
    qkv = jnp.dot(x16_ref[0], wqkv_ref[...],
                  preferred_element_type=jnp.float32)
    qkv = qkv + bqkv_ref[...]
    q = qkv[:, :D]
    k = qkv[:, D:2 * D]
    v = qkv[:, 2 * D:]
    pos = (lax.broadcasted_iota(jnp.int32, (BT, HALF), 0) + t * BT
           ).astype(jnp.float32)
    j = lax.broadcasted_iota(jnp.int32, (BT, HALF), 1).astype(jnp.float32)
    inv_freq = jnp.exp(j * (-_LN1E4 / HALF))
    f = pos * inv_freq
    c = jnp.cos(f)
    s = jnp.sin(f)

    def rope(z):
        z1 = z[:, :HALF]
        z2 = z[:, HALF:]
        return jnp.concatenate([z1 * c - z2 * s, z1 * s + z2 * c], axis=1)

    q_ref[0] = rope(q)
    k_ref[0] = rope(k)
    v_ref[0] = v
    qi_ref[0] = jnp.dot(xb, wiq_ref[...], preferred_element_type=jnp.float32)
    ki_ref[0] = jnp.dot(xb, wik_ref[...], preferred_element_type=jnp.float32)


def _scores_body(qi_ref, ki_ref, w_ref, ks_ref):
    t = pl.program_id(1)
    qib = qi_ref[0]
    kib = ki_ref[0]
    acc = jnp.zeros((1, S), jnp.float32)
    for h in range(H):
        qh = qib[:, h * HD:(h + 1) * HD]
        kh = kib[:, h * HD:(h + 1) * HD]
        sc = lax.dot_general(qh, kh, (((1,), (1,)), ((), ())),
                             preferred_element_type=jnp.float32)
        acc = acc + w_ref[h] * jnp.sum(jnp.maximum(sc, 0.0), axis=0,
                                       keepdims=True)

    @pl.when(t == 0)
    def _():
        ks_ref[0] = acc

    @pl.when(t != 0)
    def _():
        ks_ref[0] = ks_ref[0] + acc


def _cumsum_lanes(x):
    n = 1
    while n < S:
        x = x + jnp.concatenate(
            [jnp.zeros((B, n), x.dtype), x[:, :S - n]], axis=1)
        n *= 2
    return x


def _topk_body(ks_ref, idx_ref):
    ks = ks_ref[...]
    bits = lax.bitcast_convert_type(ks, jnp.int32)
    key = bits ^ ((bits >> 31) & jnp.int32(0x7FFFFFFF))
    kk = jnp.int32(K_TOP)
    nneg = jnp.sum((key >= 0).astype(jnp.int32), axis=1, keepdims=True)
    base = jnp.where(nneg >= kk, jnp.int32(0), jnp.int32(-2147483648))

    def bit_body(i, m):
        bbit = lax.shift_left(jnp.int32(1), jnp.int32(30) - i)
        t2 = base | m | bbit
        cnt = jnp.sum((key >= t2).astype(jnp.int32), axis=1, keepdims=True)
        return jnp.where(cnt >= kk, m | bbit, m)

    m = lax.fori_loop(0, 31, bit_body, jnp.zeros((B, 1), jnp.int32))
    thr = base | m
    gt = key > thr
    ties = key == thr
    need = kk - jnp.sum(gt.astype(jnp.int32), axis=1, keepdims=True)
    tie_rank = _cumsum_lanes(ties.astype(jnp.int32))
    sel = gt | (ties & (tie_rank <= need))
    rank = _cumsum_lanes(sel.astype(jnp.int32)) - 1
    posi = lax.broadcasted_iota(jnp.int32, (1, S), 1)
    pos_hl = jnp.concatenate(
        [(posi >> 8).astype(jnp.float32), (posi & 255).astype(jnp.float32)],
        axis=0)
    riota = lax.broadcasted_iota(jnp.int32, (K_TOP, S), 0)
    for b in range(B):
        selb = jnp.broadcast_to(sel[b:b + 1, :], (K_TOP, S))
        rankb = jnp.broadcast_to(rank[b:b + 1, :], (K_TOP, S))
        oh = (selb & (rankb == riota)).astype(jnp.float32)
        hl = lax.dot_general(pos_hl, oh, (((1,), (1,)), ((), ())),
                             preferred_element_type=jnp.float32)
        idx_f = hl[0:1] * 256.0 + hl[1:2]
        idx_ref[b:b + 1, :] = idx_f.astype(jnp.int32) + jnp.int32(b * S)


def _attn_body(q_ref, ksel_ref, vsel_ref, wo_ref, bo_ref, o_ref):
    qb = q_ref[0].astype(jnp.bfloat16)
    sc = lax.dot_general(qb, ksel_ref[0].astype(jnp.bfloat16),
                         (((1,), (1,)), ((), ())),
                         preferred_element_type=jnp.float32) * (1.0 / 32.0)
    mx = jnp.max(sc, axis=1, keepdims=True)
    p = jnp.exp(sc - mx)
    p = p / jnp.sum(p, axis=1, keepdims=True)
    attn = jnp.dot(p.astype(jnp.bfloat16), vsel_ref[0].astype(jnp.bfloat16),
                   preferred_element_type=jnp.float32)
    o_ref[0] = (jnp.dot(attn.astype(jnp.bfloat16), wo_ref[...],
                        preferred_element_type=jnp.float32) + bo_ref[...])


@functools.cache
def _make_sc_gather():
    def body(k_hbm, v_hbm, idx_hbm, ksel_hbm, vsel_hbm,
             idx_v, krows, vrows, sem_k, sem_v):
        wid = lax.axis_index("s") * _NC + lax.axis_index("c")
        base = wid * _RPW
        pltpu.sync_copy(idx_hbm.at[pl.ds(base, _RPW)], idx_v)
        cp_k = pltpu.async_copy(k_hbm.at[idx_v], krows, sem_k)
        cp_v = pltpu.async_copy(v_hbm.at[idx_v], vrows, sem_v)
        cp_k.wait()
        cp_v.wait()
        pltpu.sync_copy(krows, ksel_hbm.at[pl.ds(base, _RPW)])
        pltpu.sync_copy(vrows, vsel_hbm.at[pl.ds(base, _RPW)])

    return pl.kernel(
        body,
        out_type=(jax.ShapeDtypeStruct((_ROWS, D), jnp.float32),
                  jax.ShapeDtypeStruct((_ROWS, D), jnp.float32)),
        mesh=plsc.VectorSubcoreMesh(core_axis_name="c", subcore_axis_name="s",
                                    num_cores=_NC, num_subcores=_NS),
        scratch_types=[
            pltpu.VMEM((_RPW,), jnp.int32),
            pltpu.VMEM((_RPW, D), jnp.float32),
            pltpu.VMEM((_RPW, D), jnp.float32),
            pltpu.SemaphoreType.DMA,
            pltpu.SemaphoreType.DMA,
        ],
    )


def _proj(x, Wqkv, bqkv, Wiq, Wik):
    return pl.pallas_call(
        _proj_body,
        grid=(B, S // BT),
        in_specs=[
            pl.BlockSpec((1, BT, D), lambda b, t: (b, t, 0)),
            pl.BlockSpec((1, BT, D), lambda b, t: (b, t, 0)),
            pl.BlockSpec((D, 3 * D), lambda b, t: (0, 0)),
            pl.BlockSpec((1, 3 * D), lambda b, t: (0, 0)),
            pl.BlockSpec((D, IDX_D), lambda b, t: (0, 0)),
            pl.BlockSpec((D, IDX_D), lambda b, t: (0, 0)),
        ],
        out_specs=[
            pl.BlockSpec((1, BT, D), lambda b, t: (b, t, 0)),
            pl.BlockSpec((1, BT, D), lambda b, t: (b, t, 0)),
            pl.BlockSpec((1, BT, D), lambda b, t: (b, t, 0)),
            pl.BlockSpec((1, BT, IDX_D), lambda b, t: (b, t, 0)),
            pl.BlockSpec((1, BT, IDX_D), lambda b, t: (b, t, 0)),
        ],
        out_shape=[
            jax.ShapeDtypeStruct((B, S, D), jnp.float32),
            jax.ShapeDtypeStruct((B, S, D), jnp.float32),
            jax.ShapeDtypeStruct((B, S, D), jnp.float32),
            jax.ShapeDtypeStruct((B, S, IDX_D), jnp.float32),
            jax.ShapeDtypeStruct((B, S, IDX_D), jnp.float32),
        ],
    )(x.astype(jnp.bfloat16), x, Wqkv.astype(jnp.bfloat16),
      bqkv.reshape(1, 3 * D), Wiq, Wik)


def _scores(qi, ki, w_head):
    return pl.pallas_call(
        _scores_body,
        grid=(B, S // BT),
        in_specs=[
            pl.BlockSpec((1, BT, IDX_D), lambda b, t: (b, t, 0)),
            pl.BlockSpec((1, S, IDX_D), lambda b, t: (b, 0, 0)),
            pl.BlockSpec(memory_space=pltpu.SMEM),
        ],
        out_specs=pl.BlockSpec((1, 1, S), lambda b, t: (b, 0, 0)),
        out_shape=jax.ShapeDtypeStruct((B, 1, S), jnp.float32),
    )(qi, ki, w_head).reshape(B, S)


def _topk(ks):
    return pl.pallas_call(
        _topk_body,
        out_shape=jax.ShapeDtypeStruct((B, K_TOP), jnp.int32),
    )(ks)


def _attn(q, ksel, vsel, Wo, bo):
    return pl.pallas_call(
        _attn_body,
        grid=(B, S // BT),
        in_specs=[
            pl.BlockSpec((1, BT, D), lambda b, t: (b, t, 0)),
            pl.BlockSpec((1, K_TOP, D), lambda b, t: (b, 0, 0)),
            pl.BlockSpec((1, K_TOP, D), lambda b, t: (b, 0, 0)),
            pl.BlockSpec((D, D), lambda b, t: (0, 0)),
            pl.BlockSpec((1, D), lambda b, t: (0, 0)),
        ],
        out_specs=pl.BlockSpec((1, BT, D), lambda b, t: (b, t, 0)),
        out_shape=jax.ShapeDtypeStruct((B, S, D), jnp.float32),
    )(q, ksel.reshape(B, K_TOP, D), vsel.reshape(B, K_TOP, D),
      Wo.astype(jnp.bfloat16), bo.reshape(1, D))


def kernel(x, Wqkv, bqkv, Wo, bo, Wiq, Wik, w_head):
    q, k, v, qi, ki = _proj(x, Wqkv, bqkv, Wiq, Wik)
    ks = _scores(qi, ki, w_head)
    idx = _topk(ks)
    ksel, vsel = _make_sc_gather()(k.reshape(B * S, D), v.reshape(B * S, D),
                                   idx.reshape(_ROWS))
    return _attn(q, ksel, vsel, Wo, bo)


# speedup vs baseline: 1.0521x; 1.0521x over previous
import functools

import jax
import jax.numpy as jnp
from jax import lax
from jax.experimental import pallas as pl
from jax.experimental.pallas import tpu as pltpu
from jax.experimental.pallas import tpu_sc as plsc

B = 2
S = 2048
D = 1024
H = 4
HD = 64
IDX_D = H * HD
K_TOP = 256
HALF = D // 2
BT = 512
_LN1E4 = 9.210340371976184

_NC = 2
_NS = 16
_NW = _NC * _NS
_ROWS = B * K_TOP
_RPW = _ROWS // _NW


def _rope_tables(pos, n):
    j = lax.broadcasted_iota(jnp.int32, (n, HALF), 1).astype(jnp.float32)
    inv_freq = jnp.exp(j * (-_LN1E4 / HALF))
    f = pos * inv_freq
    return jnp.cos(f), jnp.sin(f)


def _rope_apply(z, c, s):
    z1 = z[:, :HALF]
    z2 = z[:, HALF:]
    return jnp.concatenate([z1 * c - z2 * s, z1 * s + z2 * c], axis=1)


def _idxproj_body(x_ref, wiq_ref, wik_ref, qi_ref, ki_ref):
    xb = x_ref[0]
    qi_ref[0] = jnp.dot(xb, wiq_ref[...], preferred_element_type=jnp.float32)
    ki_ref[0] = jnp.dot(xb, wik_ref[...], preferred_element_type=jnp.float32)


def _idxproj(x, Wiq, Wik):
    return pl.pallas_call(
        _idxproj_body,
        grid=(B, S // BT),
        in_specs=[
            pl.BlockSpec((1, BT, D), lambda b, t: (b, t, 0)),
            pl.BlockSpec((D, IDX_D), lambda b, t: (0, 0)),
            pl.BlockSpec((D, IDX_D), lambda b, t: (0, 0)),
        ],
        out_specs=[
            pl.BlockSpec((1, BT, IDX_D), lambda b, t: (b, t, 0)),
            pl.BlockSpec((1, BT, IDX_D), lambda b, t: (b, t, 0)),
        ],
        out_shape=[
            jax.ShapeDtypeStruct((B, S, IDX_D), jnp.float32),
            jax.ShapeDtypeStruct((B, S, IDX_D), jnp.float32),
        ],
    )(x, Wiq, Wik)


def _scores_body(qi_ref, ki_ref, w_ref, ks_ref):
    t = pl.program_id(1)
    qib = qi_ref[0]
    kib = ki_ref[0]
    acc = jnp.zeros((1, S), jnp.float32)
    for h in range(H):
        qh = qib[:, h * HD:(h + 1) * HD]
        kh = kib[:, h * HD:(h + 1) * HD]
        sc = lax.dot_general(qh, kh, (((1,), (1,)), ((), ())),
                             preferred_element_type=jnp.float32)
        acc = acc + w_ref[h] * jnp.sum(jnp.maximum(sc, 0.0), axis=0,
                                       keepdims=True)

    @pl.when(t == 0)
    def _():
        ks_ref[0] = acc

    @pl.when(t != 0)
    def _():
        ks_ref[0] = ks_ref[0] + acc


def _scores(qi, ki, w_head):
    return pl.pallas_call(
        _scores_body,
        grid=(B, S // BT),
        in_specs=[
            pl.BlockSpec((1, BT, IDX_D), lambda b, t: (b, t, 0)),
            pl.BlockSpec((1, S, IDX_D), lambda b, t: (b, 0, 0)),
            pl.BlockSpec(memory_space=pltpu.SMEM),
        ],
        out_specs=pl.BlockSpec((1, 1, S), lambda b, t: (b, 0, 0)),
        out_shape=jax.ShapeDtypeStruct((B, 1, S), jnp.float32),
    )(qi, ki, w_head).reshape(B, S)


def _cumsum_lanes(x):
    n = 1
    while n < S:
        x = x + jnp.concatenate(
            [jnp.zeros((B, n), x.dtype), x[:, :S - n]], axis=1)
        n *= 2
    return x


def _topk_body(ks_ref, idx_ref, pos_ref):
    ks = ks_ref[...]
    bits = lax.bitcast_convert_type(ks, jnp.int32)
    key = bits ^ ((bits >> 31) & jnp.int32(0x7FFFFFFF))
    kk = jnp.int32(K_TOP)
    nneg = jnp.sum((key >= 0).astype(jnp.int32), axis=1, keepdims=True)
    base = jnp.where(nneg >= kk, jnp.int32(0), jnp.int32(-2147483648))

    def bit_body(i, m):
        bbit = lax.shift_left(jnp.int32(1), jnp.int32(30) - i)
        t2 = base | m | bbit
        cnt = jnp.sum((key >= t2).astype(jnp.int32), axis=1, keepdims=True)
        return jnp.where(cnt >= kk, m | bbit, m)

    m = lax.fori_loop(0, 31, bit_body, jnp.zeros((B, 1), jnp.int32))
    thr = base | m
    gt = key > thr
    ties = key == thr
    need = kk - jnp.sum(gt.astype(jnp.int32), axis=1, keepdims=True)
    tie_rank = _cumsum_lanes(ties.astype(jnp.int32))
    sel = gt | (ties & (tie_rank <= need))
    rank = _cumsum_lanes(sel.astype(jnp.int32)) - 1
    posi = lax.broadcasted_iota(jnp.int32, (1, S), 1)
    pos_hl = jnp.concatenate(
        [(posi >> 8).astype(jnp.float32), (posi & 255).astype(jnp.float32)],
        axis=0)
    riota = lax.broadcasted_iota(jnp.int32, (K_TOP, S), 0)
    for b in range(B):
        selb = jnp.broadcast_to(sel[b:b + 1, :], (K_TOP, S))
        rankb = jnp.broadcast_to(rank[b:b + 1, :], (K_TOP, S))
        oh = (selb & (rankb == riota)).astype(jnp.float32)
        hl = lax.dot_general(pos_hl, oh, (((1,), (1,)), ((), ())),
                             preferred_element_type=jnp.float32)
        idx_f = hl[0:1] * 256.0 + hl[1:2]
        pos_ref[b:b + 1, :] = idx_f
        idx_ref[b:b + 1, :] = idx_f.astype(jnp.int32) + jnp.int32(b * S)


def _topk(ks):
    return pl.pallas_call(
        _topk_body,
        out_shape=[jax.ShapeDtypeStruct((B, K_TOP), jnp.int32),
                   jax.ShapeDtypeStruct((B, K_TOP), jnp.float32)],
    )(ks)


@functools.cache
def _make_sc_gather():
    def body(x_hbm, idx_hbm, xsel_hbm, idx_v, rows, sem):
        wid = lax.axis_index("s") * _NC + lax.axis_index("c")
        base = wid * _RPW
        pltpu.sync_copy(idx_hbm.at[pl.ds(base, _RPW)], idx_v)
        pltpu.async_copy(x_hbm.at[idx_v], rows, sem).wait()
        pltpu.sync_copy(rows, xsel_hbm.at[pl.ds(base, _RPW)])

    return pl.kernel(
        body,
        out_type=jax.ShapeDtypeStruct((_ROWS, D), jnp.float32),
        mesh=plsc.VectorSubcoreMesh(core_axis_name="c", subcore_axis_name="s",
                                    num_cores=_NC, num_subcores=_NS),
        scratch_types=[
            pltpu.VMEM((_RPW,), jnp.int32),
            pltpu.VMEM((_RPW, D), jnp.float32),
            pltpu.SemaphoreType.DMA,
        ],
    )


def _kvsel_body(xsel_ref, pos_ref, wkv_ref, bkv_ref, ksel_ref, vsel_ref):
    xs = xsel_ref[0].astype(jnp.bfloat16)
    kv = jnp.dot(xs, wkv_ref[...], preferred_element_type=jnp.float32)
    kv = kv + bkv_ref[...]
    c, s = _rope_tables(pos_ref[0], K_TOP)
    ksel_ref[0] = _rope_apply(kv[:, :D], c, s).astype(jnp.bfloat16)
    vsel_ref[0] = kv[:, D:].astype(jnp.bfloat16)


def _kvsel(xsel, pos, Wkv, bkv):
    return pl.pallas_call(
        _kvsel_body,
        grid=(B,),
        in_specs=[
            pl.BlockSpec((1, K_TOP, D), lambda b: (b, 0, 0)),
            pl.BlockSpec((1, K_TOP, 1), lambda b: (b, 0, 0)),
            pl.BlockSpec((D, 2 * D), lambda b: (0, 0)),
            pl.BlockSpec((1, 2 * D), lambda b: (0, 0)),
        ],
        out_specs=[
            pl.BlockSpec((1, K_TOP, D), lambda b: (b, 0, 0)),
            pl.BlockSpec((1, K_TOP, D), lambda b: (b, 0, 0)),
        ],
        out_shape=[
            jax.ShapeDtypeStruct((B, K_TOP, D), jnp.bfloat16),
            jax.ShapeDtypeStruct((B, K_TOP, D), jnp.bfloat16),
        ],
    )(xsel, pos, Wkv, bkv)


def _attn_body(x16_ref, wq_ref, bq_ref, ksel_ref, vsel_ref, wo_ref, bo_ref,
               o_ref):
    t = pl.program_id(1)
    q = jnp.dot(x16_ref[0], wq_ref[...], preferred_element_type=jnp.float32)
    q = q + bq_ref[...]
    pos = (lax.broadcasted_iota(jnp.int32, (BT, 1), 0) + t * BT
           ).astype(jnp.float32)
    c, s = _rope_tables(pos, BT)
    qb = _rope_apply(q, c, s).astype(jnp.bfloat16)
    sc = lax.dot_general(qb, ksel_ref[0], (((1,), (1,)), ((), ())),
                         preferred_element_type=jnp.float32) * (1.0 / 32.0)
    mx = jnp.max(sc, axis=1, keepdims=True)
    p = jnp.exp(sc - mx)
    p = p / jnp.sum(p, axis=1, keepdims=True)
    attn = jnp.dot(p.astype(jnp.bfloat16), vsel_ref[0],
                   preferred_element_type=jnp.float32)
    o_ref[0] = (jnp.dot(attn.astype(jnp.bfloat16), wo_ref[...],
                        preferred_element_type=jnp.float32) + bo_ref[...])


def _attn(x16, Wq, bq, ksel, vsel, Wo, bo):
    return pl.pallas_call(
        _attn_body,
        grid=(B, S // BT),
        in_specs=[
            pl.BlockSpec((1, BT, D), lambda b, t: (b, t, 0)),
            pl.BlockSpec((D, D), lambda b, t: (0, 0)),
            pl.BlockSpec((1, D), lambda b, t: (0, 0)),
            pl.BlockSpec((1, K_TOP, D), lambda b, t: (b, 0, 0)),
            pl.BlockSpec((1, K_TOP, D), lambda b, t: (b, 0, 0)),
            pl.BlockSpec((D, D), lambda b, t: (0, 0)),
            pl.BlockSpec((1, D), lambda b, t: (0, 0)),
        ],
        out_specs=pl.BlockSpec((1, BT, D), lambda b, t: (b, t, 0)),
        out_shape=jax.ShapeDtypeStruct((B, S, D), jnp.float32),
    )(x16, Wq, bq, ksel, vsel, Wo, bo)


def kernel(x, Wqkv, bqkv, Wo, bo, Wiq, Wik, w_head):
    qi, ki = _idxproj(x, Wiq, Wik)
    ks = _scores(qi, ki, w_head)
    idx, pos = _topk(ks)
    xsel = _make_sc_gather()(x.reshape(B * S, D), idx.reshape(_ROWS))
    ksel, vsel = _kvsel(xsel.reshape(B, K_TOP, D), pos.reshape(B, K_TOP, 1),
                        Wqkv[:, D:].astype(jnp.bfloat16),
                        bqkv[D:].reshape(1, 2 * D))
    return _attn(x.astype(jnp.bfloat16), Wqkv[:, :D].astype(jnp.bfloat16),
                 bqkv[:D].reshape(1, D), ksel, vsel,
                 Wo.astype(jnp.bfloat16), bo.reshape(1, D))


# speedup vs baseline: 1.0617x; 1.0091x over previous
import functools

import jax
import jax.numpy as jnp
from jax import lax
from jax.experimental import pallas as pl
from jax.experimental.pallas import tpu as pltpu
from jax.experimental.pallas import tpu_sc as plsc

B = 2
S = 2048
D = 1024
H = 4
HD = 64
IDX_D = H * HD
K_TOP = 256
HALF = D // 2
BT = 512
_LN1E4 = 9.210340371976184

_NC = 2
_NS = 16
_NW = _NC * _NS
_ROWS = B * K_TOP
_RPW = _ROWS // _NW


def _rope_tables(pos, n):
    j = lax.broadcasted_iota(jnp.int32, (n, HALF), 1).astype(jnp.float32)
    inv_freq = jnp.exp(j * (-_LN1E4 / HALF))
    f = pos * inv_freq
    return jnp.cos(f), jnp.sin(f)


def _rope_apply(z, c, s):
    z1 = z[:, :HALF]
    z2 = z[:, HALF:]
    return jnp.concatenate([z1 * c - z2 * s, z1 * s + z2 * c], axis=1)


def _idxproj_body(x_ref, wiq_ref, wik_ref, qi_ref, ki_ref):
    xb = x_ref[0]
    qi_ref[0] = jnp.dot(xb, wiq_ref[...], preferred_element_type=jnp.float32)
    ki_ref[0] = jnp.dot(xb, wik_ref[...], preferred_element_type=jnp.float32)


def _idxproj(x, Wiq, Wik):
    return pl.pallas_call(
        _idxproj_body,
        grid=(B, S // BT),
        in_specs=[
            pl.BlockSpec((1, BT, D), lambda b, t: (b, t, 0)),
            pl.BlockSpec((D, IDX_D), lambda b, t: (0, 0)),
            pl.BlockSpec((D, IDX_D), lambda b, t: (0, 0)),
        ],
        out_specs=[
            pl.BlockSpec((1, BT, IDX_D), lambda b, t: (b, t, 0)),
            pl.BlockSpec((1, BT, IDX_D), lambda b, t: (b, t, 0)),
        ],
        out_shape=[
            jax.ShapeDtypeStruct((B, S, IDX_D), jnp.float32),
            jax.ShapeDtypeStruct((B, S, IDX_D), jnp.float32),
        ],
    )(x, Wiq, Wik)


def _scores_body(qi_ref, ki_ref, w_ref, ks_ref):
    t = pl.program_id(1)
    qib = qi_ref[0]
    kib = ki_ref[0]
    acc = jnp.zeros((1, S), jnp.float32)
    for h in range(H):
        qh = qib[:, h * HD:(h + 1) * HD]
        kh = kib[:, h * HD:(h + 1) * HD]
        sc = lax.dot_general(qh, kh, (((1,), (1,)), ((), ())),
                             preferred_element_type=jnp.float32)
        acc = acc + w_ref[h] * jnp.sum(jnp.maximum(sc, 0.0), axis=0,
                                       keepdims=True)

    @pl.when(t == 0)
    def _():
        ks_ref[0] = acc

    @pl.when(t != 0)
    def _():
        ks_ref[0] = ks_ref[0] + acc


def _scores(qi, ki, w_head):
    return pl.pallas_call(
        _scores_body,
        grid=(B, S // BT),
        in_specs=[
            pl.BlockSpec((1, BT, IDX_D), lambda b, t: (b, t, 0)),
            pl.BlockSpec((1, S, IDX_D), lambda b, t: (b, 0, 0)),
            pl.BlockSpec(memory_space=pltpu.SMEM),
        ],
        out_specs=pl.BlockSpec((1, 1, S), lambda b, t: (b, 0, 0)),
        out_shape=jax.ShapeDtypeStruct((B, 1, S), jnp.float32),
    )(qi, ki, w_head).reshape(B, S)


def _cumsum_lanes(x):
    n = 1
    while n < S:
        x = x + jnp.concatenate(
            [jnp.zeros((B, n), x.dtype), x[:, :S - n]], axis=1)
        n *= 2
    return x


def _topk_body(ks_ref, idx_ref, pos_ref):
    ks = ks_ref[...]
    bits = lax.bitcast_convert_type(ks, jnp.int32)
    key = bits ^ ((bits >> 31) & jnp.int32(0x7FFFFFFF))
    kk = jnp.int32(K_TOP)
    nneg = jnp.sum((key >= 0).astype(jnp.int32), axis=1, keepdims=True)
    base = jnp.where(nneg >= kk, jnp.int32(0), jnp.int32(-2147483648))

    def bit_body(i, m):
        bbit = lax.shift_left(jnp.int32(1), jnp.int32(30) - i)
        t2 = base | m | bbit
        cnt = jnp.sum((key >= t2).astype(jnp.int32), axis=1, keepdims=True)
        return jnp.where(cnt >= kk, m | bbit, m)

    m = lax.fori_loop(0, 31, bit_body, jnp.zeros((B, 1), jnp.int32))
    thr = base | m
    gt = key > thr
    ties = key == thr
    need = kk - jnp.sum(gt.astype(jnp.int32), axis=1, keepdims=True)
    tie_rank = _cumsum_lanes(ties.astype(jnp.int32))
    sel = gt | (ties & (tie_rank <= need))
    rank = _cumsum_lanes(sel.astype(jnp.int32)) - 1
    posi = lax.broadcasted_iota(jnp.int32, (1, S), 1)
    pos_hl = jnp.concatenate(
        [(posi >> 8).astype(jnp.float32), (posi & 255).astype(jnp.float32)],
        axis=0)
    riota = lax.broadcasted_iota(jnp.int32, (K_TOP, S), 0)
    for b in range(B):
        selb = jnp.broadcast_to(sel[b:b + 1, :], (K_TOP, S))
        rankb = jnp.broadcast_to(rank[b:b + 1, :], (K_TOP, S))
        oh = (selb & (rankb == riota)).astype(jnp.float32)
        hl = lax.dot_general(pos_hl, oh, (((1,), (1,)), ((), ())),
                             preferred_element_type=jnp.float32)
        idx_f = hl[0:1] * 256.0 + hl[1:2]
        pos_ref[b:b + 1, :] = idx_f
        idx_ref[b:b + 1, :] = idx_f.astype(jnp.int32) + jnp.int32(b * S)


def _topk(ks):
    return pl.pallas_call(
        _topk_body,
        out_shape=[jax.ShapeDtypeStruct((B, K_TOP), jnp.int32),
                   jax.ShapeDtypeStruct((B, K_TOP), jnp.float32)],
    )(ks)


@functools.cache
def _make_sc_gather():
    def body(x_hbm, idx_hbm, xsel_hbm, idx_v, rows, sem):
        wid = lax.axis_index("s") * _NC + lax.axis_index("c")
        base = wid * _RPW
        pltpu.sync_copy(idx_hbm.at[pl.ds(base, _RPW)], idx_v)
        pltpu.async_copy(x_hbm.at[idx_v], rows, sem).wait()
        pltpu.sync_copy(rows, xsel_hbm.at[pl.ds(base, _RPW)])

    return pl.kernel(
        body,
        out_type=jax.ShapeDtypeStruct((_ROWS, D), jnp.float32),
        mesh=plsc.VectorSubcoreMesh(core_axis_name="c", subcore_axis_name="s",
                                    num_cores=_NC, num_subcores=_NS),
        scratch_types=[
            pltpu.VMEM((_RPW,), jnp.int32),
            pltpu.VMEM((_RPW, D), jnp.float32),
            pltpu.SemaphoreType.DMA,
        ],
    )


def _kvsel_body(xsel_ref, pos_ref, wkv_ref, bkv_ref, ksel_ref, vsel_ref):
    xs = xsel_ref[0].astype(jnp.bfloat16)
    kv = jnp.dot(xs, wkv_ref[...], preferred_element_type=jnp.float32)
    kv = kv + bkv_ref[...]
    c, s = _rope_tables(pos_ref[0], K_TOP)
    ksel_ref[0] = _rope_apply(kv[:, :D], c, s).astype(jnp.bfloat16)
    vsel_ref[0] = kv[:, D:].astype(jnp.bfloat16)


def _kvsel(xsel, pos, Wkv, bkv):
    return pl.pallas_call(
        _kvsel_body,
        grid=(B,),
        in_specs=[
            pl.BlockSpec((1, K_TOP, D), lambda b: (b, 0, 0)),
            pl.BlockSpec((1, K_TOP, 1), lambda b: (b, 0, 0)),
            pl.BlockSpec((D, 2 * D), lambda b: (0, 0)),
            pl.BlockSpec((1, 2 * D), lambda b: (0, 0)),
        ],
        out_specs=[
            pl.BlockSpec((1, K_TOP, D), lambda b: (b, 0, 0)),
            pl.BlockSpec((1, K_TOP, D), lambda b: (b, 0, 0)),
        ],
        out_shape=[
            jax.ShapeDtypeStruct((B, K_TOP, D), jnp.bfloat16),
            jax.ShapeDtypeStruct((B, K_TOP, D), jnp.bfloat16),
        ],
    )(xsel, pos, Wkv, bkv)


def _attn_body(x16_ref, wq_ref, bq_ref, xsel_ref, pos_ref, wkv_ref, bkv_ref,
               wo_ref, bo_ref, o_ref, ksel_s, vsel_s):
    t = pl.program_id(1)

    @pl.when(t == 0)
    def _():
        xs = xsel_ref[0].astype(jnp.bfloat16)
        kv = jnp.dot(xs, wkv_ref[...], preferred_element_type=jnp.float32)
        kv = kv + bkv_ref[...]
        c, s = _rope_tables(pos_ref[0], K_TOP)
        ksel_s[...] = _rope_apply(kv[:, :D], c, s).astype(jnp.bfloat16)
        vsel_s[...] = kv[:, D:].astype(jnp.bfloat16)

    q = jnp.dot(x16_ref[0], wq_ref[...], preferred_element_type=jnp.float32)
    q = q + bq_ref[...]
    pos = (lax.broadcasted_iota(jnp.int32, (BT, 1), 0) + t * BT
           ).astype(jnp.float32)
    c, s = _rope_tables(pos, BT)
    qb = _rope_apply(q, c, s).astype(jnp.bfloat16)
    sc = lax.dot_general(qb, ksel_s[...], (((1,), (1,)), ((), ())),
                         preferred_element_type=jnp.float32) * (1.0 / 32.0)
    mx = jnp.max(sc, axis=1, keepdims=True)
    p = jnp.exp(sc - mx)
    p = p / jnp.sum(p, axis=1, keepdims=True)
    attn = jnp.dot(p.astype(jnp.bfloat16), vsel_s[...],
                   preferred_element_type=jnp.float32)
    o_ref[0] = (jnp.dot(attn.astype(jnp.bfloat16), wo_ref[...],
                        preferred_element_type=jnp.float32) + bo_ref[...])


def _attn(x16, Wq, bq, xsel, pos, Wkv, bkv, Wo, bo):
    return pl.pallas_call(
        _attn_body,
        grid=(B, S // BT),
        in_specs=[
            pl.BlockSpec((1, BT, D), lambda b, t: (b, t, 0)),
            pl.BlockSpec((D, D), lambda b, t: (0, 0)),
            pl.BlockSpec((1, D), lambda b, t: (0, 0)),
            pl.BlockSpec((1, K_TOP, D), lambda b, t: (b, 0, 0)),
            pl.BlockSpec((1, K_TOP, 1), lambda b, t: (b, 0, 0)),
            pl.BlockSpec((D, 2 * D), lambda b, t: (0, 0)),
            pl.BlockSpec((1, 2 * D), lambda b, t: (0, 0)),
            pl.BlockSpec((D, D), lambda b, t: (0, 0)),
            pl.BlockSpec((1, D), lambda b, t: (0, 0)),
        ],
        out_specs=pl.BlockSpec((1, BT, D), lambda b, t: (b, t, 0)),
        out_shape=jax.ShapeDtypeStruct((B, S, D), jnp.float32),
        scratch_shapes=[
            pltpu.VMEM((K_TOP, D), jnp.bfloat16),
            pltpu.VMEM((K_TOP, D), jnp.bfloat16),
        ],
    )(x16, Wq, bq, xsel, pos, Wkv, bkv, Wo, bo)


def kernel(x, Wqkv, bqkv, Wo, bo, Wiq, Wik, w_head):
    qi, ki = _idxproj(x, Wiq, Wik)
    ks = _scores(qi, ki, w_head)
    idx, pos = _topk(ks)
    xsel = _make_sc_gather()(x.reshape(B * S, D), idx.reshape(_ROWS))
    return _attn(x.astype(jnp.bfloat16), Wqkv[:, :D].astype(jnp.bfloat16),
                 bqkv[:D].reshape(1, D), xsel.reshape(B, K_TOP, D),
                 pos.reshape(B, K_TOP, 1),
                 Wqkv[:, D:].astype(jnp.bfloat16), bqkv[D:].reshape(1, 2 * D),
                 Wo.astype(jnp.bfloat16), bo.reshape(1, D))


# speedup vs baseline: 1.1110x; 1.0464x over previous
import functools

import jax
import jax.numpy as jnp
from jax import lax
from jax.experimental import pallas as pl
from jax.experimental.pallas import tpu as pltpu
from jax.experimental.pallas import tpu_sc as plsc

B = 2
S = 2048
D = 1024
H = 4
HD = 64
IDX_D = H * HD
K_TOP = 256
HALF = D // 2
BT = 512
_LN1E4 = 9.210340371976184

_NC = 2
_NS = 16
_NW = _NC * _NS
_ROWS = B * K_TOP
_RPW = _ROWS // _NW


def _rope_tables(pos, n):
    j = lax.broadcasted_iota(jnp.int32, (n, HALF), 1).astype(jnp.float32)
    inv_freq = jnp.exp(j * (-_LN1E4 / HALF))
    f = pos * inv_freq
    return jnp.cos(f), jnp.sin(f)


def _rope_apply(z, c, s):
    z1 = z[:, :HALF]
    z2 = z[:, HALF:]
    return jnp.concatenate([z1 * c - z2 * s, z1 * s + z2 * c], axis=1)


def _cumsum_lanes(x):
    n = 1
    while n < S:
        x = x + jnp.concatenate(
            [jnp.zeros((B, n), x.dtype), x[:, :S - n]], axis=1)
        n *= 2
    return x


def _topk_write(ks, idx_ref, pos_ref):
    bits = lax.bitcast_convert_type(ks, jnp.int32)
    key = bits ^ ((bits >> 31) & jnp.int32(0x7FFFFFFF))
    kk = jnp.int32(K_TOP)
    nneg = jnp.sum((key >= 0).astype(jnp.int32), axis=1, keepdims=True)
    base = jnp.where(nneg >= kk, jnp.int32(0), jnp.int32(-2147483648))

    def bit_body(i, m):
        bbit = lax.shift_left(jnp.int32(1), jnp.int32(30) - i)
        t2 = base | m | bbit
        cnt = jnp.sum((key >= t2).astype(jnp.int32), axis=1, keepdims=True)
        return jnp.where(cnt >= kk, m | bbit, m)

    m = lax.fori_loop(0, 31, bit_body, jnp.zeros((B, 1), jnp.int32))
    thr = base | m
    gt = key > thr
    ties = key == thr
    need = kk - jnp.sum(gt.astype(jnp.int32), axis=1, keepdims=True)
    tie_rank = _cumsum_lanes(ties.astype(jnp.int32))
    sel = gt | (ties & (tie_rank <= need))
    rank = _cumsum_lanes(sel.astype(jnp.int32)) - 1
    posi = lax.broadcasted_iota(jnp.int32, (1, S), 1)
    pos_hl = jnp.concatenate(
        [(posi >> 8).astype(jnp.float32), (posi & 255).astype(jnp.float32)],
        axis=0)
    riota = lax.broadcasted_iota(jnp.int32, (K_TOP, S), 0)
    for b in range(B):
        selb = jnp.broadcast_to(sel[b:b + 1, :], (K_TOP, S))
        rankb = jnp.broadcast_to(rank[b:b + 1, :], (K_TOP, S))
        oh = (selb & (rankb == riota)).astype(jnp.float32)
        hl = lax.dot_general(pos_hl, oh, (((1,), (1,)), ((), ())),
                             preferred_element_type=jnp.float32)
        idx_f = hl[0:1] * 256.0 + hl[1:2]
        pos_ref[b:b + 1, :] = idx_f
        idx_ref[b:b + 1, :] = idx_f.astype(jnp.int32) + jnp.int32(b * S)


def _indexer_body(x_ref, wiq_ref, wik_ref, w_ref, idx_ref, pos_ref,
                  qibuf, kibuf, ksbuf):
    b = pl.program_id(0)
    p = pl.program_id(1)
    t = pl.program_id(2)

    @pl.when(p == 0)
    def _():
        xb = x_ref[0]
        qibuf[pl.ds(t * BT, BT), :] = jnp.dot(
            xb, wiq_ref[...], preferred_element_type=jnp.float32)
        kibuf[pl.ds(t * BT, BT), :] = jnp.dot(
            xb, wik_ref[...], preferred_element_type=jnp.float32)

    @pl.when(p == 1)
    def _():
        qib = qibuf[pl.ds(t * BT, BT), :]
        kib = kibuf[...]
        acc = jnp.zeros((1, S), jnp.float32)
        for h in range(H):
            qh = qib[:, h * HD:(h + 1) * HD]
            kh = kib[:, h * HD:(h + 1) * HD]
            sc = lax.dot_general(qh, kh, (((1,), (1,)), ((), ())),
                                 preferred_element_type=jnp.float32)
            acc = acc + w_ref[h] * jnp.sum(jnp.maximum(sc, 0.0), axis=0,
                                           keepdims=True)

        @pl.when(t == 0)
        def _():
            ksbuf[pl.ds(b, 1), :] = acc

        @pl.when(t != 0)
        def _():
            ksbuf[pl.ds(b, 1), :] = ksbuf[pl.ds(b, 1), :] + acc

        @pl.when((b == B - 1) & (t == S // BT - 1))
        def _():
            _topk_write(ksbuf[...], idx_ref, pos_ref)


def _indexer(x, Wiq, Wik, w_head):
    return pl.pallas_call(
        _indexer_body,
        grid=(B, 2, S // BT),
        in_specs=[
            pl.BlockSpec((1, BT, D), lambda b, p, t: (b, t * (1 - p), 0)),
            pl.BlockSpec((D, IDX_D), lambda b, p, t: (0, 0)),
            pl.BlockSpec((D, IDX_D), lambda b, p, t: (0, 0)),
            pl.BlockSpec(memory_space=pltpu.SMEM),
        ],
        out_specs=[
            pl.BlockSpec((B, K_TOP), lambda b, p, t: (0, 0)),
            pl.BlockSpec((B, K_TOP), lambda b, p, t: (0, 0)),
        ],
        out_shape=[
            jax.ShapeDtypeStruct((B, K_TOP), jnp.int32),
            jax.ShapeDtypeStruct((B, K_TOP), jnp.float32),
        ],
        scratch_shapes=[
            pltpu.VMEM((S, IDX_D), jnp.float32),
            pltpu.VMEM((S, IDX_D), jnp.float32),
            pltpu.VMEM((B, S), jnp.float32),
        ],
    )(x, Wiq, Wik, w_head)


@functools.cache
def _make_sc_gather():
    def body(x_hbm, idx_hbm, xsel_hbm, idx_v, rows, sem):
        wid = lax.axis_index("s") * _NC + lax.axis_index("c")
        base = wid * _RPW
        pltpu.sync_copy(idx_hbm.at[pl.ds(base, _RPW)], idx_v)
        pltpu.async_copy(x_hbm.at[idx_v], rows, sem).wait()
        pltpu.sync_copy(rows, xsel_hbm.at[pl.ds(base, _RPW)])

    return pl.kernel(
        body,
        out_type=jax.ShapeDtypeStruct((_ROWS, D), jnp.float32),
        mesh=plsc.VectorSubcoreMesh(core_axis_name="c", subcore_axis_name="s",
                                    num_cores=_NC, num_subcores=_NS),
        scratch_types=[
            pltpu.VMEM((_RPW,), jnp.int32),
            pltpu.VMEM((_RPW, D), jnp.float32),
            pltpu.SemaphoreType.DMA,
        ],
    )


def _attn_body(x16_ref, wq_ref, bq_ref, xsel_ref, pos_ref, wkv_ref, bkv_ref,
               wo_ref, bo_ref, o_ref, ksel_s, vsel_s):
    t = pl.program_id(1)

    @pl.when(t == 0)
    def _():
        xs = xsel_ref[0].astype(jnp.bfloat16)
        kv = jnp.dot(xs, wkv_ref[...], preferred_element_type=jnp.float32)
        kv = kv + bkv_ref[...]
        c, s = _rope_tables(pos_ref[0], K_TOP)
        ksel_s[...] = _rope_apply(kv[:, :D], c, s).astype(jnp.bfloat16)
        vsel_s[...] = kv[:, D:].astype(jnp.bfloat16)

    q = jnp.dot(x16_ref[0], wq_ref[...], preferred_element_type=jnp.float32)
    q = q + bq_ref[...]
    pos = (lax.broadcasted_iota(jnp.int32, (BT, 1), 0) + t * BT
           ).astype(jnp.float32)
    c, s = _rope_tables(pos, BT)
    qb = _rope_apply(q, c, s).astype(jnp.bfloat16)
    sc = lax.dot_general(qb, ksel_s[...], (((1,), (1,)), ((), ())),
                         preferred_element_type=jnp.float32) * (1.0 / 32.0)
    mx = jnp.max(sc, axis=1, keepdims=True)
    p = jnp.exp(sc - mx)
    p = p / jnp.sum(p, axis=1, keepdims=True)
    attn = jnp.dot(p.astype(jnp.bfloat16), vsel_s[...],
                   preferred_element_type=jnp.float32)
    o_ref[0] = (jnp.dot(attn.astype(jnp.bfloat16), wo_ref[...],
                        preferred_element_type=jnp.float32) + bo_ref[...])


def _attn(x16, Wq, bq, xsel, pos, Wkv, bkv, Wo, bo):
    return pl.pallas_call(
        _attn_body,
        grid=(B, S // BT),
        in_specs=[
            pl.BlockSpec((1, BT, D), lambda b, t: (b, t, 0)),
            pl.BlockSpec((D, D), lambda b, t: (0, 0)),
            pl.BlockSpec((1, D), lambda b, t: (0, 0)),
            pl.BlockSpec((1, K_TOP, D), lambda b, t: (b, 0, 0)),
            pl.BlockSpec((1, K_TOP, 1), lambda b, t: (b, 0, 0)),
            pl.BlockSpec((D, 2 * D), lambda b, t: (0, 0)),
            pl.BlockSpec((1, 2 * D), lambda b, t: (0, 0)),
            pl.BlockSpec((D, D), lambda b, t: (0, 0)),
            pl.BlockSpec((1, D), lambda b, t: (0, 0)),
        ],
        out_specs=pl.BlockSpec((1, BT, D), lambda b, t: (b, t, 0)),
        out_shape=jax.ShapeDtypeStruct((B, S, D), jnp.float32),
        scratch_shapes=[
            pltpu.VMEM((K_TOP, D), jnp.bfloat16),
            pltpu.VMEM((K_TOP, D), jnp.bfloat16),
        ],
    )(x16, Wq, bq, xsel, pos, Wkv, bkv, Wo, bo)


def kernel(x, Wqkv, bqkv, Wo, bo, Wiq, Wik, w_head):
    idx, pos = _indexer(x, Wiq, Wik, w_head)
    xsel = _make_sc_gather()(x.reshape(B * S, D), idx.reshape(_ROWS))
    return _attn(x.astype(jnp.bfloat16), Wqkv[:, :D].astype(jnp.bfloat16),
                 bqkv[:D].reshape(1, D), xsel.reshape(B, K_TOP, D),
                 pos.reshape(B, K_TOP, 1),
                 Wqkv[:, D:].astype(jnp.bfloat16), bqkv[D:].reshape(1, 2 * D),
                 Wo.astype(jnp.bfloat16), bo.reshape(1, D))


# speedup vs baseline: 1.1272x; 1.0146x over previous
import functools

import jax
import jax.numpy as jnp
from jax import lax
from jax.experimental import pallas as pl
from jax.experimental.pallas import tpu as pltpu
from jax.experimental.pallas import tpu_sc as plsc

B = 2
S = 2048
D = 1024
H = 4
HD = 64
IDX_D = H * HD
K_TOP = 256
HALF = D // 2
BT = 512
_LN1E4 = 9.210340371976184

_NC = 2
_NS = 16
_NW = _NC * _NS
_ROWS = B * K_TOP
_RPW = _ROWS // _NW


def _rope_tables(pos, n):
    j = lax.broadcasted_iota(jnp.int32, (n, HALF), 1).astype(jnp.float32)
    inv_freq = jnp.exp(j * (-_LN1E4 / HALF))
    f = pos * inv_freq
    return jnp.cos(f), jnp.sin(f)


def _rope_apply(z, c, s):
    z1 = z[:, :HALF]
    z2 = z[:, HALF:]
    return jnp.concatenate([z1 * c - z2 * s, z1 * s + z2 * c], axis=1)


def _cumsum_lanes(x):
    n = 1
    while n < S:
        x = x + jnp.concatenate(
            [jnp.zeros((B, n), x.dtype), x[:, :S - n]], axis=1)
        n *= 2
    return x


def _topk_write(ks, idx_ref, pos_ref):
    bits = lax.bitcast_convert_type(ks, jnp.int32)
    key = bits ^ ((bits >> 31) & jnp.int32(0x7FFFFFFF))
    kk = jnp.int32(K_TOP)
    nneg = jnp.sum((key >= 0).astype(jnp.int32), axis=1, keepdims=True)
    base = jnp.where(nneg >= kk, jnp.int32(0), jnp.int32(-2147483648))

    def bit_body(i, m):
        bbit = lax.shift_left(jnp.int32(1), jnp.int32(30) - i)
        t2 = base | m | bbit
        cnt = jnp.sum((key >= t2).astype(jnp.int32), axis=1, keepdims=True)
        return jnp.where(cnt >= kk, m | bbit, m)

    m = lax.fori_loop(0, 31, bit_body, jnp.zeros((B, 1), jnp.int32))
    thr = base | m
    gt = key > thr
    ties = key == thr
    need = kk - jnp.sum(gt.astype(jnp.int32), axis=1, keepdims=True)
    tie_rank = _cumsum_lanes(ties.astype(jnp.int32))
    sel = gt | (ties & (tie_rank <= need))
    rank = _cumsum_lanes(sel.astype(jnp.int32)) - 1
    posi = lax.broadcasted_iota(jnp.int32, (1, S), 1)
    pos_hl = jnp.concatenate(
        [(posi >> 8).astype(jnp.float32), (posi & 255).astype(jnp.float32)],
        axis=0)
    riota = lax.broadcasted_iota(jnp.int32, (K_TOP, S), 0)
    for b in range(B):
        selb = jnp.broadcast_to(sel[b:b + 1, :], (K_TOP, S))
        rankb = jnp.broadcast_to(rank[b:b + 1, :], (K_TOP, S))
        oh = (selb & (rankb == riota)).astype(jnp.float32)
        hl = lax.dot_general(pos_hl, oh, (((1,), (1,)), ((), ())),
                             preferred_element_type=jnp.float32)
        idx_f = hl[0:1] * 256.0 + hl[1:2]
        pos_ref[b:b + 1, :] = idx_f
        idx_ref[b:b + 1, :] = idx_f.astype(jnp.int32) + jnp.int32(b * S)


def _indexer_body(x_ref, wiq_ref, wik_ref, w_ref, idx_ref, pos_ref,
                  qibuf, kibuf, ksbuf):
    b = pl.program_id(0)
    p = pl.program_id(1)
    t = pl.program_id(2)

    @pl.when(p == 0)
    def _():
        xb = x_ref[0]
        qibuf[pl.ds(t * BT, BT), :] = jnp.dot(
            xb, wiq_ref[...], preferred_element_type=jnp.float32)
        kibuf[pl.ds(t * BT, BT), :] = jnp.dot(
            xb, wik_ref[...], preferred_element_type=jnp.float32)

    @pl.when(p == 1)
    def _():
        qib = qibuf[pl.ds(t * BT, BT), :]
        kib = kibuf[...]
        acc = jnp.zeros((1, S), jnp.float32)
        for h in range(H):
            qh = qib[:, h * HD:(h + 1) * HD]
            kh = kib[:, h * HD:(h + 1) * HD]
            sc = lax.dot_general(qh, kh, (((1,), (1,)), ((), ())),
                                 preferred_element_type=jnp.float32)
            acc = acc + w_ref[h] * jnp.sum(jnp.maximum(sc, 0.0), axis=0,
                                           keepdims=True)

        @pl.when(t == 0)
        def _():
            ksbuf[pl.ds(b, 1), :] = acc

        @pl.when(t != 0)
        def _():
            ksbuf[pl.ds(b, 1), :] = ksbuf[pl.ds(b, 1), :] + acc

        @pl.when((b == B - 1) & (t == S // BT - 1))
        def _():
            _topk_write(ksbuf[...], idx_ref, pos_ref)


def _indexer(x, Wiq, Wik, w_head):
    return pl.pallas_call(
        _indexer_body,
        grid=(B, 2, S // BT),
        in_specs=[
            pl.BlockSpec((1, BT, D), lambda b, p, t: (b, t * (1 - p), 0)),
            pl.BlockSpec((D, IDX_D), lambda b, p, t: (0, 0)),
            pl.BlockSpec((D, IDX_D), lambda b, p, t: (0, 0)),
            pl.BlockSpec(memory_space=pltpu.SMEM),
        ],
        out_specs=[
            pl.BlockSpec((B, K_TOP), lambda b, p, t: (0, 0)),
            pl.BlockSpec((B, K_TOP), lambda b, p, t: (0, 0)),
        ],
        out_shape=[
            jax.ShapeDtypeStruct((B, K_TOP), jnp.int32),
            jax.ShapeDtypeStruct((B, K_TOP), jnp.float32),
        ],
        scratch_shapes=[
            pltpu.VMEM((S, IDX_D), jnp.float32),
            pltpu.VMEM((S, IDX_D), jnp.float32),
            pltpu.VMEM((B, S), jnp.float32),
        ],
    )(x, Wiq, Wik, w_head)


@functools.cache
def _make_sc_gather():
    def body(x_hbm, idx_hbm, xsel_hbm, idx_v, rows, sem):
        wid = lax.axis_index("s") * _NC + lax.axis_index("c")
        base = wid * _RPW
        pltpu.sync_copy(idx_hbm.at[pl.ds(base, _RPW)], idx_v)
        pltpu.async_copy(x_hbm.at[idx_v], rows, sem).wait()
        pltpu.sync_copy(rows, xsel_hbm.at[pl.ds(base, _RPW)])

    return pl.kernel(
        body,
        out_type=jax.ShapeDtypeStruct((_ROWS, D), jnp.float32),
        mesh=plsc.VectorSubcoreMesh(core_axis_name="c", subcore_axis_name="s",
                                    num_cores=_NC, num_subcores=_NS),
        scratch_types=[
            pltpu.VMEM((_RPW,), jnp.int32),
            pltpu.VMEM((_RPW, D), jnp.float32),
            pltpu.SemaphoreType.DMA,
        ],
    )


BT_A = 1024


def _attn_body(x16_ref, wq_ref, bq_ref, xsel_ref, pos_ref, wkv_ref, bkv_ref,
               wo_ref, bo_ref, o_ref, ksel_s, vsel_s):
    t = pl.program_id(1)

    @pl.when(t == 0)
    def _():
        xs = xsel_ref[0].astype(jnp.bfloat16)
        kv = jnp.dot(xs, wkv_ref[...], preferred_element_type=jnp.float32)
        kv = kv + bkv_ref[...]
        c, s = _rope_tables(pos_ref[0], K_TOP)
        ksel_s[...] = _rope_apply(kv[:, :D], c, s).astype(jnp.bfloat16)
        vsel_s[...] = kv[:, D:].astype(jnp.bfloat16)

    q = jnp.dot(x16_ref[0], wq_ref[...], preferred_element_type=jnp.float32)
    q = q + bq_ref[...]
    pos = (lax.broadcasted_iota(jnp.int32, (BT_A, 1), 0) + t * BT_A
           ).astype(jnp.float32)
    c, s = _rope_tables(pos, BT_A)
    qb = _rope_apply(q, c, s).astype(jnp.bfloat16)
    sc = lax.dot_general(qb, ksel_s[...], (((1,), (1,)), ((), ())),
                         preferred_element_type=jnp.float32) * (1.0 / 32.0)
    mx = jnp.max(sc, axis=1, keepdims=True)
    p = jnp.exp(sc - mx)
    p = p / jnp.sum(p, axis=1, keepdims=True)
    attn = jnp.dot(p.astype(jnp.bfloat16), vsel_s[...],
                   preferred_element_type=jnp.float32)
    o_ref[0] = (jnp.dot(attn.astype(jnp.bfloat16), wo_ref[...],
                        preferred_element_type=jnp.float32) + bo_ref[...])


def _attn(x16, Wq, bq, xsel, pos, Wkv, bkv, Wo, bo):
    return pl.pallas_call(
        _attn_body,
        grid=(B, S // BT_A),
        in_specs=[
            pl.BlockSpec((1, BT_A, D), lambda b, t: (b, t, 0)),
            pl.BlockSpec((D, D), lambda b, t: (0, 0)),
            pl.BlockSpec((1, D), lambda b, t: (0, 0)),
            pl.BlockSpec((1, K_TOP, D), lambda b, t: (b, 0, 0)),
            pl.BlockSpec((1, K_TOP, 1), lambda b, t: (b, 0, 0)),
            pl.BlockSpec((D, 2 * D), lambda b, t: (0, 0)),
            pl.BlockSpec((1, 2 * D), lambda b, t: (0, 0)),
            pl.BlockSpec((D, D), lambda b, t: (0, 0)),
            pl.BlockSpec((1, D), lambda b, t: (0, 0)),
        ],
        out_specs=pl.BlockSpec((1, BT_A, D), lambda b, t: (b, t, 0)),
        out_shape=jax.ShapeDtypeStruct((B, S, D), jnp.float32),
        scratch_shapes=[
            pltpu.VMEM((K_TOP, D), jnp.bfloat16),
            pltpu.VMEM((K_TOP, D), jnp.bfloat16),
        ],
    )(x16, Wq, bq, xsel, pos, Wkv, bkv, Wo, bo)


def kernel(x, Wqkv, bqkv, Wo, bo, Wiq, Wik, w_head):
    idx, pos = _indexer(x, Wiq, Wik, w_head)
    xsel = _make_sc_gather()(x.reshape(B * S, D), idx.reshape(_ROWS))
    return _attn(x.astype(jnp.bfloat16), Wqkv[:, :D].astype(jnp.bfloat16),
                 bqkv[:D].reshape(1, D), xsel.reshape(B, K_TOP, D),
                 pos.reshape(B, K_TOP, 1),
                 Wqkv[:, D:].astype(jnp.bfloat16), bqkv[D:].reshape(1, 2 * D),
                 Wo.astype(jnp.bfloat16), bo.reshape(1, D))


# speedup vs baseline: 1.1493x; 1.0196x over previous
import functools

import jax
import jax.numpy as jnp
from jax import lax
from jax.experimental import pallas as pl
from jax.experimental.pallas import tpu as pltpu
from jax.experimental.pallas import tpu_sc as plsc

B = 2
S = 2048
D = 1024
H = 4
HD = 64
IDX_D = H * HD
K_TOP = 256
HALF = D // 2
BT = 1024
_LN1E4 = 9.210340371976184

_NC = 2
_NS = 16
_NW = _NC * _NS
_ROWS = B * K_TOP
_RPW = _ROWS // _NW


def _rope_tables(pos, n):
    j = lax.broadcasted_iota(jnp.int32, (n, HALF), 1).astype(jnp.float32)
    inv_freq = jnp.exp(j * (-_LN1E4 / HALF))
    f = pos * inv_freq
    return jnp.cos(f), jnp.sin(f)


def _rope_apply(z, c, s):
    z1 = z[:, :HALF]
    z2 = z[:, HALF:]
    return jnp.concatenate([z1 * c - z2 * s, z1 * s + z2 * c], axis=1)


def _cumsum_lanes(x):
    n = 1
    while n < S:
        x = x + jnp.concatenate(
            [jnp.zeros((B, n), x.dtype), x[:, :S - n]], axis=1)
        n *= 2
    return x


def _topk_write(ks, idx_ref, pos_ref):
    bits = lax.bitcast_convert_type(ks, jnp.int32)
    key = bits ^ ((bits >> 31) & jnp.int32(0x7FFFFFFF))
    kk = jnp.int32(K_TOP)
    nneg = jnp.sum((key >= 0).astype(jnp.int32), axis=1, keepdims=True)
    base = jnp.where(nneg >= kk, jnp.int32(0), jnp.int32(-2147483648))

    def bit_body(i, m):
        bbit = lax.shift_left(jnp.int32(1), jnp.int32(30) - i)
        t2 = base | m | bbit
        cnt = jnp.sum((key >= t2).astype(jnp.int32), axis=1, keepdims=True)
        return jnp.where(cnt >= kk, m | bbit, m)

    m = lax.fori_loop(0, 31, bit_body, jnp.zeros((B, 1), jnp.int32))
    thr = base | m
    gt = key > thr
    ties = key == thr
    need = kk - jnp.sum(gt.astype(jnp.int32), axis=1, keepdims=True)
    tie_rank = _cumsum_lanes(ties.astype(jnp.int32))
    sel = gt | (ties & (tie_rank <= need))
    rank = _cumsum_lanes(sel.astype(jnp.int32)) - 1
    posi = lax.broadcasted_iota(jnp.int32, (1, S), 1)
    pos_hl = jnp.concatenate(
        [(posi >> 8).astype(jnp.float32), (posi & 255).astype(jnp.float32)],
        axis=0)
    riota = lax.broadcasted_iota(jnp.int32, (K_TOP, S), 0)
    for b in range(B):
        selb = jnp.broadcast_to(sel[b:b + 1, :], (K_TOP, S))
        rankb = jnp.broadcast_to(rank[b:b + 1, :], (K_TOP, S))
        oh = (selb & (rankb == riota)).astype(jnp.float32)
        hl = lax.dot_general(pos_hl, oh, (((1,), (1,)), ((), ())),
                             preferred_element_type=jnp.float32)
        idx_f = hl[0:1] * 256.0 + hl[1:2]
        pos_ref[b:b + 1, :] = idx_f
        idx_ref[b:b + 1, :] = idx_f.astype(jnp.int32) + jnp.int32(b * S)


def _indexer_body(x_ref, wiq_ref, wik_ref, w_ref, idx_ref, pos_ref,
                  qibuf, kibuf, ksbuf):
    b = pl.program_id(0)
    p = pl.program_id(1)
    t = pl.program_id(2)

    @pl.when(p == 0)
    def _():
        xb = x_ref[0]
        qibuf[pl.ds(t * BT, BT), :] = jnp.dot(
            xb, wiq_ref[...], preferred_element_type=jnp.float32)
        kibuf[pl.ds(t * BT, BT), :] = jnp.dot(
            xb, wik_ref[...], preferred_element_type=jnp.float32)

    @pl.when(p == 1)
    def _():
        qib = qibuf[pl.ds(t * BT, BT), :]
        kib = kibuf[...]
        acc = jnp.zeros((1, S), jnp.float32)
        for h in range(H):
            qh = qib[:, h * HD:(h + 1) * HD]
            kh = kib[:, h * HD:(h + 1) * HD]
            sc = lax.dot_general(qh, kh, (((1,), (1,)), ((), ())),
                                 preferred_element_type=jnp.float32)
            acc = acc + w_ref[h] * jnp.sum(jnp.maximum(sc, 0.0), axis=0,
                                           keepdims=True)

        @pl.when(t == 0)
        def _():
            ksbuf[pl.ds(b, 1), :] = acc

        @pl.when(t != 0)
        def _():
            ksbuf[pl.ds(b, 1), :] = ksbuf[pl.ds(b, 1), :] + acc

        @pl.when((b == B - 1) & (t == S // BT - 1))
        def _():
            _topk_write(ksbuf[...], idx_ref, pos_ref)


def _indexer(x, Wiq, Wik, w_head):
    return pl.pallas_call(
        _indexer_body,
        grid=(B, 2, S // BT),
        in_specs=[
            pl.BlockSpec((1, BT, D), lambda b, p, t: (b, t * (1 - p), 0)),
            pl.BlockSpec((D, IDX_D), lambda b, p, t: (0, 0)),
            pl.BlockSpec((D, IDX_D), lambda b, p, t: (0, 0)),
            pl.BlockSpec(memory_space=pltpu.SMEM),
        ],
        out_specs=[
            pl.BlockSpec((B, K_TOP), lambda b, p, t: (0, 0)),
            pl.BlockSpec((B, K_TOP), lambda b, p, t: (0, 0)),
        ],
        out_shape=[
            jax.ShapeDtypeStruct((B, K_TOP), jnp.int32),
            jax.ShapeDtypeStruct((B, K_TOP), jnp.float32),
        ],
        scratch_shapes=[
            pltpu.VMEM((S, IDX_D), jnp.float32),
            pltpu.VMEM((S, IDX_D), jnp.float32),
            pltpu.VMEM((B, S), jnp.float32),
        ],
    )(x, Wiq, Wik, w_head)


@functools.cache
def _make_sc_gather():
    def body(x_hbm, idx_hbm, xsel_hbm, idx_v, rows, sem):
        wid = lax.axis_index("s") * _NC + lax.axis_index("c")
        base = wid * _RPW
        pltpu.sync_copy(idx_hbm.at[pl.ds(base, _RPW)], idx_v)
        pltpu.async_copy(x_hbm.at[idx_v], rows, sem).wait()
        pltpu.sync_copy(rows, xsel_hbm.at[pl.ds(base, _RPW)])

    return pl.kernel(
        body,
        out_type=jax.ShapeDtypeStruct((_ROWS, D), jnp.float32),
        mesh=plsc.VectorSubcoreMesh(core_axis_name="c", subcore_axis_name="s",
                                    num_cores=_NC, num_subcores=_NS),
        scratch_types=[
            pltpu.VMEM((_RPW,), jnp.int32),
            pltpu.VMEM((_RPW, D), jnp.float32),
            pltpu.SemaphoreType.DMA,
        ],
    )


BT_A = 1024


def _attn_body(x16_ref, wq_ref, bq_ref, xsel_ref, pos_ref, wkv_ref, bkv_ref,
               wo_ref, bo_ref, o_ref, ksel_s, vsel_s):
    t = pl.program_id(1)

    @pl.when(t == 0)
    def _():
        xs = xsel_ref[0].astype(jnp.bfloat16)
        kv = jnp.dot(xs, wkv_ref[...], preferred_element_type=jnp.float32)
        kv = kv + bkv_ref[...]
        c, s = _rope_tables(pos_ref[0], K_TOP)
        ksel_s[...] = _rope_apply(kv[:, :D], c, s).astype(jnp.bfloat16)
        vsel_s[...] = kv[:, D:].astype(jnp.bfloat16)

    q = jnp.dot(x16_ref[0], wq_ref[...], preferred_element_type=jnp.float32)
    q = q + bq_ref[...]
    pos = (lax.broadcasted_iota(jnp.int32, (BT_A, 1), 0) + t * BT_A
           ).astype(jnp.float32)
    c, s = _rope_tables(pos, BT_A)
    qb = _rope_apply(q, c, s).astype(jnp.bfloat16)
    sc = lax.dot_general(qb, ksel_s[...], (((1,), (1,)), ((), ())),
                         preferred_element_type=jnp.float32) * (1.0 / 32.0)
    mx = jnp.max(sc, axis=1, keepdims=True)
    p = jnp.exp(sc - mx)
    p = p / jnp.sum(p, axis=1, keepdims=True)
    attn = jnp.dot(p.astype(jnp.bfloat16), vsel_s[...],
                   preferred_element_type=jnp.float32)
    o_ref[0] = (jnp.dot(attn.astype(jnp.bfloat16), wo_ref[...],
                        preferred_element_type=jnp.float32) + bo_ref[...])


def _attn(x16, Wq, bq, xsel, pos, Wkv, bkv, Wo, bo):
    return pl.pallas_call(
        _attn_body,
        grid=(B, S // BT_A),
        in_specs=[
            pl.BlockSpec((1, BT_A, D), lambda b, t: (b, t, 0)),
            pl.BlockSpec((D, D), lambda b, t: (0, 0)),
            pl.BlockSpec((1, D), lambda b, t: (0, 0)),
            pl.BlockSpec((1, K_TOP, D), lambda b, t: (b, 0, 0)),
            pl.BlockSpec((1, K_TOP, 1), lambda b, t: (b, 0, 0)),
            pl.BlockSpec((D, 2 * D), lambda b, t: (0, 0)),
            pl.BlockSpec((1, 2 * D), lambda b, t: (0, 0)),
            pl.BlockSpec((D, D), lambda b, t: (0, 0)),
            pl.BlockSpec((1, D), lambda b, t: (0, 0)),
        ],
        out_specs=pl.BlockSpec((1, BT_A, D), lambda b, t: (b, t, 0)),
        out_shape=jax.ShapeDtypeStruct((B, S, D), jnp.float32),
        scratch_shapes=[
            pltpu.VMEM((K_TOP, D), jnp.bfloat16),
            pltpu.VMEM((K_TOP, D), jnp.bfloat16),
        ],
    )(x16, Wq, bq, xsel, pos, Wkv, bkv, Wo, bo)


def kernel(x, Wqkv, bqkv, Wo, bo, Wiq, Wik, w_head):
    idx, pos = _indexer(x, Wiq, Wik, w_head)
    xsel = _make_sc_gather()(x.reshape(B * S, D), idx.reshape(_ROWS))
    return _attn(x.astype(jnp.bfloat16), Wqkv[:, :D].astype(jnp.bfloat16),
                 bqkv[:D].reshape(1, D), xsel.reshape(B, K_TOP, D),
                 pos.reshape(B, K_TOP, 1),
                 Wqkv[:, D:].astype(jnp.bfloat16), bqkv[D:].reshape(1, 2 * D),
                 Wo.astype(jnp.bfloat16), bo.reshape(1, D))
